# baseline (speedup 1.0000x reference)
import jax
import jax.numpy as jnp
from jax import lax
from jax.experimental import pallas as pl
from jax.experimental.pallas import tpu as pltpu

R = 24
INT = 11
WREP = 41.0
HW = 64
NPIX = HW * HW
C = 128
HID = 512
NB = 4
NT = 3
BATCH = 2
ROWS = BATCH * R * R
KEYS = R * R
NORM = float(BATCH * NPIX)
_PER_TOWER = 2 + NB * 8


def _shift_rows(x, off):
    if off == 0:
        return x
    z = jnp.zeros((abs(off), x.shape[1]), x.dtype)
    if off > 0:
        return jnp.concatenate([x[off:], z], axis=0)
    return jnp.concatenate([z, x[:off]], axis=0)


def _towers_kernel(*refs):
    keys_ref = refs[-1]
    s = lax.broadcasted_iota(jnp.int32, (ROWS, 1), 0)
    hpos = (s // R) % R
    wpos = s % R
    wt = (jnp.where(hpos == INT, WREP, 1.0)
          * jnp.where(wpos == INT, WREP, 1.0))
    b_id = s // (R * R)

    prod = None
    for t in range(NT):
        base = NT + t * _PER_TOWER
        fcw, fcb = refs[base], refs[base + 1]
        e = jnp.mean(refs[t][...], axis=1)
        x0 = jax.nn.relu(
            lax.dot_general(e, fcw[...], (((1,), (1,)), ((), ())),
                            preferred_element_type=jnp.float32)
            + fcb[...])
        x = jnp.where(b_id == 0, x0[0:1], x0[1:2])

        for k in range(NB):
            bb = base + 2 + k * 8
            w1, b1, dw9, dwb, w2, b2, bng, bnb = (refs[bb + i][...]
                                                  for i in range(8))
            mu = jnp.sum(x * wt, axis=0, keepdims=True) / NORM
            var = jnp.sum((x - mu) ** 2 * wt, axis=0, keepdims=True) / NORM
            xn = (x - mu) / jnp.sqrt(var + 1e-5)
            xn = xn * bng + bnb
            h = jax.nn.relu(
                lax.dot_general(xn, w1, (((1,), (1,)), ((), ())),
                                preferred_element_type=jnp.float32) + b1)
            h = jnp.where(wpos == 0, _shift_rows(h, R - 1), _shift_rows(h, -1))
            acc = jnp.zeros_like(h)
            for ky in range(3):
                for kx in range(3):
                    dy, dx = ky - 1, kx - 1
                    m = (((hpos + dy) >= 0) & ((hpos + dy) < R)
                         & ((wpos + dx) >= 0) & ((wpos + dx) < R)
                         ).astype(h.dtype)
                    kv = dw9[3 * ky + kx:3 * ky + kx + 1]
                    acc = acc + _shift_rows(h, dy * R + dx) * m * kv
            h = jax.nn.relu(acc + dwb)
            x = (x
                 + lax.dot_general(h, w2, (((1,), (1,)), ((), ())),
                                   preferred_element_type=jnp.float32) + b2)
        prod = x if prod is None else prod * x

    keys_ref[...] = prod / (jnp.sqrt(jnp.sum(prod * prod, axis=1,
                                             keepdims=True)) + 1e-6)


def _towers(args):
    return pl.pallas_call(
        _towers_kernel,
        out_shape=jax.ShapeDtypeStruct((ROWS, C), jnp.float32),
    )(*args)


def _main_kernel(v_ref, kt_ref, tvw1_ref, tvb1_ref, tvw2_ref, tvb2_ref,
                 ttw1_ref, ttb1_ref, ttw2_ref, ttb2_ref, o_ref):
    vf = v_ref[0]
    kt = kt_ref[0]

    pn = vf / (jnp.sqrt(jnp.sum(vf * vf, axis=1, keepdims=True)) + 1e-6)
    sim = lax.dot_general(pn, kt, (((1,), (0,)), ((), ())),
                          preferred_element_type=jnp.float32)
    kn2 = jnp.sum(kt * kt, axis=0, keepdims=True)
    d2 = kn2 - 2.0 * sim
    m = jnp.min(d2, axis=1, keepdims=True)
    ji = lax.broadcasted_iota(jnp.int32, d2.shape, 1)
    idx = jnp.min(jnp.where(d2 == m, ji, KEYS), axis=1, keepdims=True)
    onehot = (ji == idx).astype(jnp.float32)
    tf = lax.dot_general(onehot, kt, (((1,), (1,)), ((), ())),
                         preferred_element_type=jnp.float32)

    s = lax.broadcasted_iota(jnp.int32, (NPIX, 1), 0)
    hpos = s // HW
    wpos = s % HW

    def blend_coef(vcur, stride, pos):
        tprev = _shift_rows(tf, -stride)
        num = jnp.sum(vcur * tprev, axis=1, keepdims=True)
        den = jnp.maximum(
            jnp.sqrt(jnp.sum(vcur * vcur, axis=1, keepdims=True))
            * jnp.sqrt(jnp.sum(tprev * tprev, axis=1, keepdims=True)), 1e-8)
        return jnp.where(pos == 0, 0.0, jnp.exp(-(1.0 - num / den)))

    def linscan(vcur, stride, pos):
        A = blend_coef(vcur, stride, pos)
        Bv = (1.0 - A) * vcur
        k = 1
        while k < HW:
            live = pos >= k
            Ash = jnp.where(live, _shift_rows(A, -k * stride), 1.0)
            Bsh = jnp.where(live, _shift_rows(Bv, -k * stride), 0.0)
            Bv = A * Bsh + Bv
            A = A * Ash
            k *= 2
        return Bv

    vr = linscan(vf, 1, wpos)
    vc = linscan(vr, HW, hpos)

    def cosd(a, b):
        num = jnp.sum(a * b, axis=1, keepdims=True)
        den = jnp.maximum(
            jnp.sqrt(jnp.sum(a * a, axis=1, keepdims=True))
            * jnp.sqrt(jnp.sum(b * b, axis=1, keepdims=True)), 1e-8)
        return 1.0 - num / den

    d_tv = cosd(vc, tf)
    tnext = _shift_rows(tf, 1)
    d_tt = jnp.where(s == NPIX - 1, 0.0, cosd(tf, tnext))

    def mlp(d, w1, b1, w2, b2):
        h = jax.nn.relu(d * w1 + b1)
        return jnp.sum(h * w2, axis=1, keepdims=True) + b2

    gate = jax.nn.sigmoid(
        mlp(d_tv, tvw1_ref[...], tvb1_ref[...], tvw2_ref[...], tvb2_ref[...])
        + mlp(d_tt, ttw1_ref[...], ttb1_ref[...], ttw2_ref[...], ttb2_ref[...]))
    o_ref[0] = vc * gate


def _main(vn, keys_t, mlp_params):
    vec = lambda: pl.BlockSpec((1, 256), lambda b: (0, 0))
    scl = lambda: pl.BlockSpec((1, 1), lambda b: (0, 0))
    return pl.pallas_call(
        _main_kernel,
        grid=(BATCH,),
        in_specs=[
            pl.BlockSpec((1, NPIX, C), lambda b: (b, 0, 0)),
            pl.BlockSpec((1, C, KEYS), lambda b: (b, 0, 0)),
            vec(), vec(), vec(), scl(), vec(), vec(), vec(), scl(),
        ],
        out_specs=pl.BlockSpec((1, NPIX, C), lambda b: (b, 0, 0)),
        out_shape=jax.ShapeDtypeStruct((BATCH, NPIX, C), jnp.float32),
        compiler_params=pltpu.CompilerParams(
            dimension_semantics=("parallel",)),
    )(vn, keys_t, *mlp_params)


def kernel(V, tA, tB, tAB, params):
    targs = [tA, tB, tAB]
    for name in ('tA', 'tB', 'tAB'):
        p = params[name]
        targs += [p['fc_w'], p['fc_b'].reshape(1, C)]
        for b in p['blocks']:
            targs += [
                b['w1'], b['b1'].reshape(1, HID),
                b['dw'].reshape(HID, 9).T, b['dwb'].reshape(1, HID),
                b['w2'], b['b2'].reshape(1, C),
                b['bn_g'].reshape(1, C), b['bn_b'].reshape(1, C),
            ]

    keys = _towers(targs)
    keys_t = keys.reshape(BATCH, KEYS, C).transpose(0, 2, 1)

    vn = jnp.transpose(V, (0, 2, 3, 1)).reshape(BATCH, NPIX, C)
    mlp_params = (
        params['tv']['w1'].reshape(1, 256), params['tv']['b1'].reshape(1, 256),
        params['tv']['w2'].reshape(1, 256), params['tv']['b2'].reshape(1, 1),
        params['tt']['w1'].reshape(1, 256), params['tt']['b1'].reshape(1, 256),
        params['tt']['w2'].reshape(1, 256), params['tt']['b2'].reshape(1, 1),
    )
    out = _main(vn, keys_t, mlp_params)
    return jnp.transpose(out.reshape(BATCH, HW, HW, C), (0, 3, 1, 2))


# speedup vs baseline: 1.0650x; 1.0650x over previous
import jax
import jax.numpy as jnp
from jax import lax
from jax.experimental import pallas as pl

R = 24
INT = 11
WREP = 41.0
HW = 64
NPIX = HW * HW
C = 128
HID = 512
NB = 4
NT = 3
BATCH = 2
ROWS = BATCH * R * R
KEYS = R * R
NORM = float(BATCH * NPIX)


def _shift_rows(x, off):
    if off == 0:
        return x
    z = jnp.zeros((abs(off), x.shape[1]), x.dtype)
    if off > 0:
        return jnp.concatenate([x[off:], z], axis=0)
    return jnp.concatenate([z, x[:off]], axis=0)


def _towers_kernel(temb_ref, fcw_ref, fcb_ref, w1_ref, b1_ref, dw_ref,
                   dwb_ref, w2_ref, b2_ref, bng_ref, bnb_ref, keys_ref):
    s = lax.broadcasted_iota(jnp.int32, (ROWS, 1), 0)
    hpos = (s // R) % R
    wpos = s % R
    wt = (jnp.where(hpos == INT, WREP, 1.0)
          * jnp.where(wpos == INT, WREP, 1.0))
    b_id = s // (R * R)

    prod = None
    for t in range(NT):
        e = jnp.mean(temb_ref[t], axis=1)
        x0 = jax.nn.relu(
            lax.dot_general(e, fcw_ref[t], (((1,), (1,)), ((), ())),
                            preferred_element_type=jnp.float32)
            + fcb_ref[t:t + 1])
        x = jnp.where(b_id == 0, x0[0:1], x0[1:2])

        for k in range(NB):
            mu = jnp.sum(x * wt, axis=0, keepdims=True) / NORM
            var = jnp.sum((x - mu) ** 2 * wt, axis=0, keepdims=True) / NORM
            xn = (x - mu) / jnp.sqrt(var + 1e-5)
            xn = xn * bng_ref[t, k:k + 1] + bnb_ref[t, k:k + 1]
            h = jax.nn.relu(
                lax.dot_general(xn, w1_ref[t, k], (((1,), (1,)), ((), ())),
                                preferred_element_type=jnp.float32)
                + b1_ref[t, k:k + 1])
            h = jnp.where(wpos == 0, _shift_rows(h, R - 1), _shift_rows(h, -1))
            acc = jnp.zeros_like(h)
            for ky in range(3):
                for kx in range(3):
                    dy, dx = ky - 1, kx - 1
                    m = (((hpos + dy) >= 0) & ((hpos + dy) < R)
                         & ((wpos + dx) >= 0) & ((wpos + dx) < R)
                         ).astype(h.dtype)
                    kv = dw_ref[t, k, 3 * ky + kx:3 * ky + kx + 1]
                    acc = acc + _shift_rows(h, dy * R + dx) * m * kv
            h = jax.nn.relu(acc + dwb_ref[t, k:k + 1])
            x = (x
                 + lax.dot_general(h, w2_ref[t, k], (((1,), (1,)), ((), ())),
                                   preferred_element_type=jnp.float32)
                 + b2_ref[t, k:k + 1])
        prod = x if prod is None else prod * x

    keys_ref[...] = prod / (jnp.sqrt(jnp.sum(prod * prod, axis=1,
                                             keepdims=True)) + 1e-6)


def _towers(temb, fcw, fcb, w1, b1, dw, dwb, w2, b2, bng, bnb):
    return pl.pallas_call(
        _towers_kernel,
        out_shape=jax.ShapeDtypeStruct((ROWS, C), jnp.float32),
    )(temb, fcw, fcb, w1, b1, dw, dwb, w2, b2, bng, bnb)


def _knn_kernel(v_ref, kt_ref, tr_ref):
    v = v_ref[0]
    kt = kt_ref[0]
    pn = v / (jnp.sqrt(jnp.sum(v * v, axis=1, keepdims=True)) + 1e-6)
    sim = lax.dot_general(pn, kt, (((1,), (0,)), ((), ())),
                          preferred_element_type=jnp.float32)
    kn2 = jnp.sum(kt * kt, axis=0, keepdims=True)
    d2 = kn2 - 2.0 * sim
    m = jnp.min(d2, axis=1, keepdims=True)
    ji = lax.broadcasted_iota(jnp.int32, d2.shape, 1)
    idx = jnp.min(jnp.where(d2 == m, ji, KEYS), axis=1, keepdims=True)
    onehot = (ji == idx).astype(jnp.float32)
    tr_ref[0] = lax.dot_general(onehot, kt, (((1,), (1,)), ((), ())),
                                preferred_element_type=jnp.float32)


def _knn(vn, keys_t):
    return pl.pallas_call(
        _knn_kernel,
        grid=(BATCH,),
        in_specs=[
            pl.BlockSpec((1, NPIX, C), lambda b: (b, 0, 0)),
            pl.BlockSpec((1, C, KEYS), lambda b: (b, 0, 0)),
        ],
        out_specs=pl.BlockSpec((1, NPIX, C), lambda b: (b, 0, 0)),
        out_shape=jax.ShapeDtypeStruct((BATCH, NPIX, C), jnp.float32),
    )(vn, keys_t)


def _scanfin_kernel(v_ref, t_ref, tvw1_ref, tvb1_ref, tvw2_ref, tvb2_ref,
                    ttw1_ref, ttb1_ref, ttw2_ref, ttb2_ref, o_ref):
    vf = v_ref[0]
    tf = t_ref[0]
    s = lax.broadcasted_iota(jnp.int32, (NPIX, 1), 0)
    hpos = s // HW
    wpos = s % HW

    def blend_coef(vcur, stride, pos):
        tprev = _shift_rows(tf, -stride)
        num = jnp.sum(vcur * tprev, axis=1, keepdims=True)
        den = jnp.maximum(
            jnp.sqrt(jnp.sum(vcur * vcur, axis=1, keepdims=True))
            * jnp.sqrt(jnp.sum(tprev * tprev, axis=1, keepdims=True)), 1e-8)
        return jnp.where(pos == 0, 0.0, jnp.exp(-(1.0 - num / den)))

    def linscan(vcur, stride, pos):
        A = blend_coef(vcur, stride, pos)
        Bv = (1.0 - A) * vcur
        k = 1
        while k < HW:
            live = pos >= k
            Ash = jnp.where(live, _shift_rows(A, -k * stride), 1.0)
            Bsh = jnp.where(live, _shift_rows(Bv, -k * stride), 0.0)
            Bv = A * Bsh + Bv
            A = A * Ash
            k *= 2
        return Bv

    vr = linscan(vf, 1, wpos)
    vc = linscan(vr, HW, hpos)

    def cosd(a, b):
        num = jnp.sum(a * b, axis=1, keepdims=True)
        den = jnp.maximum(
            jnp.sqrt(jnp.sum(a * a, axis=1, keepdims=True))
            * jnp.sqrt(jnp.sum(b * b, axis=1, keepdims=True)), 1e-8)
        return 1.0 - num / den

    d_tv = cosd(vc, tf)
    tnext = _shift_rows(tf, 1)
    d_tt = jnp.where(s == NPIX - 1, 0.0, cosd(tf, tnext))

    def mlp(d, w1, b1, w2, b2):
        h = jax.nn.relu(d * w1 + b1)
        return jnp.sum(h * w2, axis=1, keepdims=True) + b2

    gate = jax.nn.sigmoid(
        mlp(d_tv, tvw1_ref[...], tvb1_ref[...], tvw2_ref[...], tvb2_ref[...])
        + mlp(d_tt, ttw1_ref[...], ttb1_ref[...], ttw2_ref[...], ttb2_ref[...]))
    o_ref[0] = vc * gate


def _scanfin(vn, tr, mlp_params):
    vec = lambda: pl.BlockSpec((1, 256), lambda b: (0, 0))
    scl = lambda: pl.BlockSpec((1, 1), lambda b: (0, 0))
    return pl.pallas_call(
        _scanfin_kernel,
        grid=(BATCH,),
        in_specs=[
            pl.BlockSpec((1, NPIX, C), lambda b: (b, 0, 0)),
            pl.BlockSpec((1, NPIX, C), lambda b: (b, 0, 0)),
            vec(), vec(), vec(), scl(), vec(), vec(), vec(), scl(),
        ],
        out_specs=pl.BlockSpec((1, NPIX, C), lambda b: (b, 0, 0)),
        out_shape=jax.ShapeDtypeStruct((BATCH, NPIX, C), jnp.float32),
    )(vn, tr, *mlp_params)


def kernel(V, tA, tB, tAB, params):
    towers = [params[n] for n in ('tA', 'tB', 'tAB')]
    temb = jnp.stack([tA, tB, tAB])
    fcw = jnp.stack([p['fc_w'] for p in towers])
    fcb = jnp.stack([p['fc_b'] for p in towers])

    def blk(name):
        return jnp.stack([jnp.stack([b[name] for b in p['blocks']])
                          for p in towers])

    w1, b1, dwb = blk('w1'), blk('b1'), blk('dwb')
    w2, b2 = blk('w2'), blk('b2')
    bng, bnb = blk('bn_g'), blk('bn_b')
    dw = blk('dw').reshape(NT, NB, HID, 9).transpose(0, 1, 3, 2)

    keys = _towers(temb, fcw, fcb, w1, b1, dw, dwb, w2, b2, bng, bnb)
    keys_t = keys.reshape(BATCH, KEYS, C).transpose(0, 2, 1)

    vn = jnp.transpose(V, (0, 2, 3, 1)).reshape(BATCH, NPIX, C)
    tr = _knn(vn, keys_t)

    mlp_params = (
        params['tv']['w1'].reshape(1, 256), params['tv']['b1'].reshape(1, 256),
        params['tv']['w2'].reshape(1, 256), params['tv']['b2'].reshape(1, 1),
        params['tt']['w1'].reshape(1, 256), params['tt']['b1'].reshape(1, 256),
        params['tt']['w2'].reshape(1, 256), params['tt']['b2'].reshape(1, 1),
    )
    out = _scanfin(vn, tr, mlp_params)
    return jnp.transpose(out.reshape(BATCH, HW, HW, C), (0, 3, 1, 2))


# speedup vs baseline: 1.3930x; 1.3080x over previous
import jax
import jax.numpy as jnp
from jax import lax
from jax.experimental import pallas as pl

R = 16
INT = 8
WREP = 49.0
HW = 64
NPIX = HW * HW
C = 128
HID = 512
NB = 4
NT = 3
BATCH = 2
ROWS = BATCH * R * R
KEYS = R * R
NORM = float(BATCH * NPIX)


def _shift_rows(x, off):
    if off == 0:
        return x
    z = jnp.zeros((abs(off), x.shape[1]), x.dtype)
    if off > 0:
        return jnp.concatenate([x[off:], z], axis=0)
    return jnp.concatenate([z, x[:off]], axis=0)


def _towers_kernel(temb_ref, fcw_ref, fcb_ref, w1_ref, b1_ref, dw_ref,
                   dwb_ref, w2_ref, b2_ref, bng_ref, bnb_ref, keys_ref):
    s = lax.broadcasted_iota(jnp.int32, (ROWS, 1), 0)
    hpos = (s // R) % R
    wpos = s % R
    wt = (jnp.where(hpos == INT, WREP, 1.0)
          * jnp.where(wpos == INT, WREP, 1.0))
    b_id = s // (R * R)

    prod = None
    for t in range(NT):
        e = jnp.mean(temb_ref[t], axis=1)
        x0 = jax.nn.relu(
            lax.dot_general(e, fcw_ref[t], (((1,), (1,)), ((), ())),
                            preferred_element_type=jnp.float32)
            + fcb_ref[t:t + 1])
        x = jnp.where(b_id == 0, x0[0:1], x0[1:2])

        for k in range(NB):
            mu = jnp.sum(x * wt, axis=0, keepdims=True) / NORM
            var = jnp.sum((x - mu) ** 2 * wt, axis=0, keepdims=True) / NORM
            xn = (x - mu) / jnp.sqrt(var + 1e-5)
            xn = xn * bng_ref[t, k:k + 1] + bnb_ref[t, k:k + 1]
            h = jax.nn.relu(
                lax.dot_general(xn, w1_ref[t, k], (((1,), (1,)), ((), ())),
                                preferred_element_type=jnp.float32)
                + b1_ref[t, k:k + 1])
            h = jnp.where(wpos == 0, _shift_rows(h, R - 1), _shift_rows(h, -1))
            acc = jnp.zeros_like(h)
            for ky in range(3):
                for kx in range(3):
                    dy, dx = ky - 1, kx - 1
                    m = (((hpos + dy) >= 0) & ((hpos + dy) < R)
                         & ((wpos + dx) >= 0) & ((wpos + dx) < R)
                         ).astype(h.dtype)
                    kv = dw_ref[t, k, 3 * ky + kx:3 * ky + kx + 1]
                    acc = acc + _shift_rows(h, dy * R + dx) * m * kv
            h = jax.nn.relu(acc + dwb_ref[t, k:k + 1])
            x = (x
                 + lax.dot_general(h, w2_ref[t, k], (((1,), (1,)), ((), ())),
                                   preferred_element_type=jnp.float32)
                 + b2_ref[t, k:k + 1])
        prod = x if prod is None else prod * x

    keys_ref[...] = prod / (jnp.sqrt(jnp.sum(prod * prod, axis=1,
                                             keepdims=True)) + 1e-6)


def _towers(temb, fcw, fcb, w1, b1, dw, dwb, w2, b2, bng, bnb):
    return pl.pallas_call(
        _towers_kernel,
        out_shape=jax.ShapeDtypeStruct((ROWS, C), jnp.float32),
    )(temb, fcw, fcb, w1, b1, dw, dwb, w2, b2, bng, bnb)


def _knn_kernel(v_ref, k_ref, tr_ref):
    v = v_ref[0]
    keys = k_ref[0]
    pn = v / (jnp.sqrt(jnp.sum(v * v, axis=1, keepdims=True)) + 1e-6)
    kn2 = jnp.sum(keys * keys, axis=1, keepdims=True)
    keys_aug = jnp.concatenate([keys, kn2], axis=1)
    pn_aug = jnp.concatenate(
        [pn * -2.0, jnp.ones((pn.shape[0], 1), jnp.float32)], axis=1)
    d2 = lax.dot_general(pn_aug, keys_aug, (((1,), (1,)), ((), ())),
                         preferred_element_type=jnp.float32)
    m = jnp.min(d2, axis=1, keepdims=True)
    ji = lax.broadcasted_iota(jnp.int32, d2.shape, 1)
    idx = jnp.min(jnp.where(d2 == m, ji, KEYS), axis=1, keepdims=True)
    onehot = (ji == idx).astype(jnp.float32)
    tr_ref[0] = lax.dot_general(onehot, keys, (((1,), (0,)), ((), ())),
                                preferred_element_type=jnp.float32)


def _knn(vn, keys):
    return pl.pallas_call(
        _knn_kernel,
        grid=(BATCH,),
        in_specs=[
            pl.BlockSpec((1, NPIX, C), lambda b: (b, 0, 0)),
            pl.BlockSpec((1, KEYS, C), lambda b: (b, 0, 0)),
        ],
        out_specs=pl.BlockSpec((1, NPIX, C), lambda b: (b, 0, 0)),
        out_shape=jax.ShapeDtypeStruct((BATCH, NPIX, C), jnp.float32),
    )(vn, keys)


def _scanfin_kernel(v_ref, t_ref, tvw1_ref, tvb1_ref, tvw2_ref, tvb2_ref,
                    ttw1_ref, ttb1_ref, ttw2_ref, ttb2_ref, o_ref):
    vf = v_ref[0]
    tf = t_ref[0]
    s = lax.broadcasted_iota(jnp.int32, (NPIX, 1), 0)
    hpos = s // HW
    wpos = s % HW

    def blend_coef(vcur, stride, pos):
        tprev = _shift_rows(tf, -stride)
        num = jnp.sum(vcur * tprev, axis=1, keepdims=True)
        den = jnp.maximum(
            jnp.sqrt(jnp.sum(vcur * vcur, axis=1, keepdims=True))
            * jnp.sqrt(jnp.sum(tprev * tprev, axis=1, keepdims=True)), 1e-8)
        return jnp.where(pos == 0, 0.0, jnp.exp(-(1.0 - num / den)))

    def linscan(vcur, stride, pos):
        A = blend_coef(vcur, stride, pos)
        Bv = (1.0 - A) * vcur
        k = 1
        while k < HW:
            live = pos >= k
            Ash = jnp.where(live, _shift_rows(A, -k * stride), 1.0)
            Bsh = jnp.where(live, _shift_rows(Bv, -k * stride), 0.0)
            Bv = A * Bsh + Bv
            A = A * Ash
            k *= 2
        return Bv

    vr = linscan(vf, 1, wpos)
    vc = linscan(vr, HW, hpos)

    def cosd(a, b):
        num = jnp.sum(a * b, axis=1, keepdims=True)
        den = jnp.maximum(
            jnp.sqrt(jnp.sum(a * a, axis=1, keepdims=True))
            * jnp.sqrt(jnp.sum(b * b, axis=1, keepdims=True)), 1e-8)
        return 1.0 - num / den

    d_tv = cosd(vc, tf)
    tnext = _shift_rows(tf, 1)
    d_tt = jnp.where(s == NPIX - 1, 0.0, cosd(tf, tnext))

    def mlp(d, w1, b1, w2, b2):
        h = jax.nn.relu(d * w1 + b1)
        return jnp.sum(h * w2, axis=1, keepdims=True) + b2

    gate = jax.nn.sigmoid(
        mlp(d_tv, tvw1_ref[...], tvb1_ref[...], tvw2_ref[...], tvb2_ref[...])
        + mlp(d_tt, ttw1_ref[...], ttb1_ref[...], ttw2_ref[...], ttb2_ref[...]))
    o_ref[0] = vc * gate


def _scanfin(vn, tr, mlp_params):
    vec = lambda: pl.BlockSpec((1, 256), lambda b: (0, 0))
    scl = lambda: pl.BlockSpec((1, 1), lambda b: (0, 0))
    return pl.pallas_call(
        _scanfin_kernel,
        grid=(BATCH,),
        in_specs=[
            pl.BlockSpec((1, NPIX, C), lambda b: (b, 0, 0)),
            pl.BlockSpec((1, NPIX, C), lambda b: (b, 0, 0)),
            vec(), vec(), vec(), scl(), vec(), vec(), vec(), scl(),
        ],
        out_specs=pl.BlockSpec((1, NPIX, C), lambda b: (b, 0, 0)),
        out_shape=jax.ShapeDtypeStruct((BATCH, NPIX, C), jnp.float32),
    )(vn, tr, *mlp_params)


def kernel(V, tA, tB, tAB, params):
    towers = [params[n] for n in ('tA', 'tB', 'tAB')]
    temb = jnp.stack([tA, tB, tAB])
    fcw = jnp.stack([p['fc_w'] for p in towers])
    fcb = jnp.stack([p['fc_b'] for p in towers])

    def blk(name):
        return jnp.stack([jnp.stack([b[name] for b in p['blocks']])
                          for p in towers])

    w1, b1, dwb = blk('w1'), blk('b1'), blk('dwb')
    w2, b2 = blk('w2'), blk('b2')
    bng, bnb = blk('bn_g'), blk('bn_b')
    dw = blk('dw').reshape(NT, NB, HID, 9).transpose(0, 1, 3, 2)

    keys = _towers(temb, fcw, fcb, w1, b1, dw, dwb, w2, b2, bng, bnb)
    keys = keys.reshape(BATCH, KEYS, C)

    vn = jnp.transpose(V, (0, 2, 3, 1)).reshape(BATCH, NPIX, C)
    tr = _knn(vn, keys)

    mlp_params = (
        params['tv']['w1'].reshape(1, 256), params['tv']['b1'].reshape(1, 256),
        params['tv']['w2'].reshape(1, 256), params['tv']['b2'].reshape(1, 1),
        params['tt']['w1'].reshape(1, 256), params['tt']['b1'].reshape(1, 256),
        params['tt']['w2'].reshape(1, 256), params['tt']['b2'].reshape(1, 1),
    )
    out = _scanfin(vn, tr, mlp_params)
    return jnp.transpose(out.reshape(BATCH, HW, HW, C), (0, 3, 1, 2))


# speedup vs baseline: 1.4198x; 1.0192x over previous
import jax
import jax.numpy as jnp
from jax import lax
from jax.experimental import pallas as pl

R = 16
INT = 8
WREP = 49.0
HW = 64
NPIX = HW * HW
C = 128
HID = 512
NB = 4
NT = 3
BATCH = 2
ROWS = BATCH * R * R
KEYS = R * R
NORM = float(BATCH * NPIX)


def _shift_rows(x, off):
    if off == 0:
        return x
    z = jnp.zeros((abs(off), x.shape[1]), x.dtype)
    if off > 0:
        return jnp.concatenate([x[off:], z], axis=0)
    return jnp.concatenate([z, x[:off]], axis=0)


def _shift4(x, d, axis):
    if d == 0:
        return x
    n = x.shape[axis]
    zshape = list(x.shape)
    zshape[axis] = abs(d)
    z = jnp.zeros(zshape, x.dtype)
    if d > 0:
        return jnp.concatenate([lax.slice_in_dim(x, d, n, axis=axis), z],
                               axis=axis)
    return jnp.concatenate([z, lax.slice_in_dim(x, 0, n + d, axis=axis)],
                           axis=axis)


def _towers_kernel(temb_ref, fcw_ref, fcb_ref, w1_ref, b1_ref, dw_ref,
                   dwb_ref, w2_ref, b2_ref, bng_ref, bnb_ref, keys_ref):
    s = lax.broadcasted_iota(jnp.int32, (ROWS, 1), 0)
    b_id = s // (R * R)
    sl = lax.broadcasted_iota(jnp.int32, (1, ROWS), 1)
    hl = (sl // R) % R
    wl = sl % R
    wt_l = (jnp.where(hl == INT, WREP, 1.0)
            * jnp.where(wl == INT, WREP, 1.0))
    hiprec = jax.lax.Precision.HIGHEST

    prod = None
    for t in range(NT):
        e = jnp.mean(temb_ref[t], axis=1)
        x0 = jax.nn.relu(
            lax.dot_general(e, fcw_ref[t], (((1,), (1,)), ((), ())),
                            preferred_element_type=jnp.float32)
            + fcb_ref[t:t + 1])
        x = jnp.where(b_id == 0, x0[0:1], x0[1:2])

        for k in range(NB):
            mu = lax.dot_general(wt_l, x, (((1,), (0,)), ((), ())),
                                 preferred_element_type=jnp.float32,
                                 precision=hiprec) / NORM
            ex2 = lax.dot_general(wt_l, x * x, (((1,), (0,)), ((), ())),
                                  preferred_element_type=jnp.float32,
                                  precision=hiprec) / NORM
            var = ex2 - mu * mu
            xn = (x - mu) / jnp.sqrt(var + 1e-5)
            xn = xn * bng_ref[t, k:k + 1] + bnb_ref[t, k:k + 1]
            h = jax.nn.relu(
                lax.dot_general(xn, w1_ref[t, k], (((1,), (1,)), ((), ())),
                                preferred_element_type=jnp.float32)
                + b1_ref[t, k:k + 1])
            h4 = h.reshape(BATCH, R, R, HID)
            h4 = jnp.concatenate([h4[:, :, R - 1:, :], h4[:, :, :R - 1, :]],
                                 axis=2)
            acc = None
            for ky in range(3):
                hy = _shift4(h4, ky - 1, 1)
                for kx in range(3):
                    kv = dw_ref[t, k, 3 * ky + kx:3 * ky + kx + 1]
                    term = _shift4(hy, kx - 1, 2) * kv
                    acc = term if acc is None else acc + term
            h = jax.nn.relu(acc + dwb_ref[t, k:k + 1]).reshape(ROWS, HID)
            x = (x
                 + lax.dot_general(h, w2_ref[t, k], (((1,), (1,)), ((), ())),
                                   preferred_element_type=jnp.float32)
                 + b2_ref[t, k:k + 1])
        prod = x if prod is None else prod * x

    keys_ref[...] = prod / (jnp.sqrt(jnp.sum(prod * prod, axis=1,
                                             keepdims=True)) + 1e-6)


def _towers(temb, fcw, fcb, w1, b1, dw, dwb, w2, b2, bng, bnb):
    return pl.pallas_call(
        _towers_kernel,
        out_shape=jax.ShapeDtypeStruct((ROWS, C), jnp.float32),
    )(temb, fcw, fcb, w1, b1, dw, dwb, w2, b2, bng, bnb)


def _knn_kernel(v_ref, k_ref, tr_ref):
    v = v_ref[0]
    keys = k_ref[0]
    pn = v / (jnp.sqrt(jnp.sum(v * v, axis=1, keepdims=True)) + 1e-6)
    kn2 = jnp.sum(keys * keys, axis=1, keepdims=True)
    keys_aug = jnp.concatenate([keys, kn2], axis=1)
    pn_aug = jnp.concatenate(
        [pn * -2.0, jnp.ones((pn.shape[0], 1), jnp.float32)], axis=1)
    d2 = lax.dot_general(pn_aug, keys_aug, (((1,), (1,)), ((), ())),
                         preferred_element_type=jnp.float32)
    m = jnp.min(d2, axis=1, keepdims=True)
    ji = lax.broadcasted_iota(jnp.int32, d2.shape, 1)
    idx = jnp.min(jnp.where(d2 == m, ji, KEYS), axis=1, keepdims=True)
    onehot = (ji == idx).astype(jnp.float32)
    tr_ref[0] = lax.dot_general(onehot, keys, (((1,), (0,)), ((), ())),
                                preferred_element_type=jnp.float32)


def _knn(vn, keys):
    return pl.pallas_call(
        _knn_kernel,
        grid=(BATCH,),
        in_specs=[
            pl.BlockSpec((1, NPIX, C), lambda b: (b, 0, 0)),
            pl.BlockSpec((1, KEYS, C), lambda b: (b, 0, 0)),
        ],
        out_specs=pl.BlockSpec((1, NPIX, C), lambda b: (b, 0, 0)),
        out_shape=jax.ShapeDtypeStruct((BATCH, NPIX, C), jnp.float32),
    )(vn, keys)


def _scanfin_kernel(v_ref, t_ref, tvw1_ref, tvb1_ref, tvw2_ref, tvb2_ref,
                    ttw1_ref, ttb1_ref, ttw2_ref, ttb2_ref, o_ref):
    vf = v_ref[0]
    tf = t_ref[0]
    s = lax.broadcasted_iota(jnp.int32, (NPIX, 1), 0)
    hpos = s // HW
    wpos = s % HW

    def blend_coef(vcur, stride, pos):
        tprev = _shift_rows(tf, -stride)
        num = jnp.sum(vcur * tprev, axis=1, keepdims=True)
        den = jnp.maximum(
            jnp.sqrt(jnp.sum(vcur * vcur, axis=1, keepdims=True))
            * jnp.sqrt(jnp.sum(tprev * tprev, axis=1, keepdims=True)), 1e-8)
        return jnp.where(pos == 0, 0.0, jnp.exp(-(1.0 - num / den)))

    def linscan(vcur, stride, pos):
        A = blend_coef(vcur, stride, pos)
        Bv = (1.0 - A) * vcur
        k = 1
        while k < HW:
            live = pos >= k
            Am = jnp.where(live, A, 0.0)
            Bv = Am * _shift_rows(Bv, -k * stride) + Bv
            A = A * jnp.where(live, _shift_rows(A, -k * stride), 1.0)
            k *= 2
        return Bv

    vr = linscan(vf, 1, wpos)
    vc = linscan(vr, HW, hpos)

    def cosd(a, b):
        num = jnp.sum(a * b, axis=1, keepdims=True)
        den = jnp.maximum(
            jnp.sqrt(jnp.sum(a * a, axis=1, keepdims=True))
            * jnp.sqrt(jnp.sum(b * b, axis=1, keepdims=True)), 1e-8)
        return 1.0 - num / den

    d_tv = cosd(vc, tf)
    tnext = _shift_rows(tf, 1)
    d_tt = jnp.where(s == NPIX - 1, 0.0, cosd(tf, tnext))

    def mlp(d, w1, b1, w2, b2):
        h = jax.nn.relu(d * w1 + b1)
        return jnp.sum(h * w2, axis=1, keepdims=True) + b2

    gate = jax.nn.sigmoid(
        mlp(d_tv, tvw1_ref[...], tvb1_ref[...], tvw2_ref[...], tvb2_ref[...])
        + mlp(d_tt, ttw1_ref[...], ttb1_ref[...], ttw2_ref[...], ttb2_ref[...]))
    o_ref[0] = vc * gate


def _scanfin(vn, tr, mlp_params):
    vec = lambda: pl.BlockSpec((1, 256), lambda b: (0, 0))
    scl = lambda: pl.BlockSpec((1, 1), lambda b: (0, 0))
    return pl.pallas_call(
        _scanfin_kernel,
        grid=(BATCH,),
        in_specs=[
            pl.BlockSpec((1, NPIX, C), lambda b: (b, 0, 0)),
            pl.BlockSpec((1, NPIX, C), lambda b: (b, 0, 0)),
            vec(), vec(), vec(), scl(), vec(), vec(), vec(), scl(),
        ],
        out_specs=pl.BlockSpec((1, NPIX, C), lambda b: (b, 0, 0)),
        out_shape=jax.ShapeDtypeStruct((BATCH, NPIX, C), jnp.float32),
    )(vn, tr, *mlp_params)


def kernel(V, tA, tB, tAB, params):
    towers = [params[n] for n in ('tA', 'tB', 'tAB')]
    temb = jnp.stack([tA, tB, tAB])
    fcw = jnp.stack([p['fc_w'] for p in towers])
    fcb = jnp.stack([p['fc_b'] for p in towers])

    def blk(name):
        return jnp.stack([jnp.stack([b[name] for b in p['blocks']])
                          for p in towers])

    w1, b1, dwb = blk('w1'), blk('b1'), blk('dwb')
    w2, b2 = blk('w2'), blk('b2')
    bng, bnb = blk('bn_g'), blk('bn_b')
    dw = blk('dw').reshape(NT, NB, HID, 9).transpose(0, 1, 3, 2)

    keys = _towers(temb, fcw, fcb, w1, b1, dw, dwb, w2, b2, bng, bnb)
    keys = keys.reshape(BATCH, KEYS, C)

    vn = jnp.transpose(V, (0, 2, 3, 1)).reshape(BATCH, NPIX, C)
    tr = _knn(vn, keys)

    mlp_params = (
        params['tv']['w1'].reshape(1, 256), params['tv']['b1'].reshape(1, 256),
        params['tv']['w2'].reshape(1, 256), params['tv']['b2'].reshape(1, 1),
        params['tt']['w1'].reshape(1, 256), params['tt']['b1'].reshape(1, 256),
        params['tt']['w2'].reshape(1, 256), params['tt']['b2'].reshape(1, 1),
    )
    out = _scanfin(vn, tr, mlp_params)
    return jnp.transpose(out.reshape(BATCH, HW, HW, C), (0, 3, 1, 2))
